# Initial kernel scaffold; baseline (speedup 1.0000x reference)
import jax
import jax.numpy as jnp
from jax.experimental import pallas as pl


def kernel(x, table, W1, b1, W2, b2):
    raise NotImplementedError("write your pallas kernel here")


# speedup vs baseline: 8.6384x; 8.6384x over previous
import functools

import jax
import jax.numpy as jnp
from jax import lax
from jax.experimental import pallas as pl
from jax.experimental.pallas import tpu as pltpu
from jax.experimental.pallas import tpu_sc as plsc

B = 16384
SEQ = 50
VOCAB = 500
EMB = 32
H1 = 256
OUT = 10

NC = 2
NS = 16
NW = NC * NS

ROWS = B * SEQ
ROWS_PER_W = ROWS // NW
CHUNK = 128
CHUNKS_PER_W = ROWS_PER_W // CHUNK
GROUP = 8
GROUPS = CHUNKS_PER_W // GROUP


def _sc_gather(flat_idx, table):
    idx3 = flat_idx.reshape(NW, CHUNKS_PER_W, CHUNK)
    mesh = plsc.VectorSubcoreMesh(core_axis_name="c", subcore_axis_name="s")

    @functools.partial(
        pl.kernel,
        mesh=mesh,
        out_type=jax.ShapeDtypeStruct((ROWS, EMB), jnp.float32),
        scratch_types=[
            pltpu.VMEM((CHUNKS_PER_W, CHUNK), jnp.int32),
            pltpu.VMEM((GROUP * CHUNK, EMB), jnp.float32),
            pltpu.SemaphoreType.DMA,
        ],
        compiler_params=pltpu.CompilerParams(use_tc_tiling_on_sc=False),
    )
    def gather_kernel(idx_hbm, table_hbm, out_hbm, idx_v, rows_v, sem):
        wid = lax.axis_index("s") * NC + lax.axis_index("c")
        base = wid * ROWS_PER_W
        pltpu.sync_copy(idx_hbm.at[wid], idx_v)

        def group_body(g, carry):
            copies = []
            for b in range(GROUP):
                copies.append(
                    pltpu.async_copy(
                        table_hbm.at[idx_v.at[g * GROUP + b]],
                        rows_v.at[pl.ds(b * CHUNK, CHUNK)],
                        sem,
                    )
                )
            for c in copies:
                c.wait()
            pltpu.sync_copy(
                rows_v,
                out_hbm.at[pl.ds(base + g * (GROUP * CHUNK), GROUP * CHUNK)],
            )
            return carry

        lax.fori_loop(0, GROUPS, group_body, 0)

    return gather_kernel(idx3, table)


BT = 1024


def _mlp(emb_flat, W1, b1, W2, b2):
    def body(e_ref, w1_ref, b1_ref, w2_ref, b2_ref, o_ref):
        h = jnp.dot(e_ref[...], w1_ref[...], preferred_element_type=jnp.float32)
        h = jnp.maximum(h + b1_ref[...], 0.0)
        logits = jnp.dot(h, w2_ref[...], preferred_element_type=jnp.float32)
        logits = logits + b2_ref[...]
        m = jnp.max(logits, axis=-1, keepdims=True)
        e = jnp.exp(logits - m)
        o_ref[...] = e / jnp.sum(e, axis=-1, keepdims=True)

    return pl.pallas_call(
        body,
        grid=(B // BT,),
        in_specs=[
            pl.BlockSpec((BT, SEQ * EMB), lambda i: (i, 0)),
            pl.BlockSpec((SEQ * EMB, H1), lambda i: (0, 0)),
            pl.BlockSpec((1, H1), lambda i: (0, 0)),
            pl.BlockSpec((H1, OUT), lambda i: (0, 0)),
            pl.BlockSpec((1, OUT), lambda i: (0, 0)),
        ],
        out_specs=pl.BlockSpec((BT, OUT), lambda i: (i, 0)),
        out_shape=jax.ShapeDtypeStruct((B, OUT), jnp.float32),
    )(emb_flat, W1, b1.reshape(1, H1), W2, b2.reshape(1, OUT))


def kernel(x, table, W1, b1, W2, b2):
    flat_idx = x.reshape(-1).astype(jnp.int32)
    emb = _sc_gather(flat_idx, table)
    emb_flat = emb.reshape(B, SEQ * EMB)
    return _mlp(emb_flat, W1, b1, W2, b2)
